# Initial kernel scaffold; baseline (speedup 1.0000x reference)
import jax
import jax.numpy as jnp
from jax.experimental import pallas as pl


def kernel(features, edge_index, W1, b1, W2, b2):
    raise NotImplementedError("write your pallas kernel here")


# speedup vs baseline: 8.0146x; 8.0146x over previous
import functools

import jax
import jax.numpy as jnp
from jax import lax
from jax.experimental import pallas as pl
from jax.experimental.pallas import tpu as pltpu
from jax.experimental.pallas import tpu_sc as plsc

N = 10000
E = 320000
NC = 2
NS = 16
NW = NC * NS
C = 80
EPW = E // NW
NCHUNK = EPW // C
NPAD = 10240

_mesh = plsc.VectorSubcoreMesh(core_axis_name="c", subcore_axis_name="s")
_sc_params = pltpu.CompilerParams(use_tc_tiling_on_sc=False)


@functools.partial(
    pl.kernel,
    out_type=jax.ShapeDtypeStruct((NC, 2, 1, NPAD), jnp.float32),
    mesh=_mesh,
    compiler_params=_sc_params,
    scratch_types=[
        pltpu.VMEM((NCHUNK, C), jnp.int32),
        pltpu.VMEM((NCHUNK, C), jnp.int32),
        pltpu.VMEM((C,), jnp.float32),
        pltpu.VMEM((NPAD // NS,), jnp.float32),
        pltpu.VMEM_SHARED((NPAD,), jnp.float32),
        pltpu.VMEM_SHARED((NPAD,), jnp.float32),
    ],
)
def _deg_kernel(src_hbm, dst_hbm, out_hbm, sidx, didx, ones_v, zer_v,
                acc_out, acc_in):
    cid = lax.axis_index("c")
    sid = lax.axis_index("s")
    wid = sid * NC + cid
    zslab = NPAD // NS

    for t in range(C // 16):
        ones_v[pl.ds(t * 16, 16)] = jnp.ones((16,), jnp.float32)
    for t in range(zslab // 16):
        zer_v[pl.ds(t * 16, 16)] = jnp.zeros((16,), jnp.float32)

    pltpu.sync_copy(src_hbm.at[wid], sidx)
    pltpu.sync_copy(dst_hbm.at[wid], didx)
    pltpu.sync_copy(zer_v, acc_out.at[pl.ds(sid * zslab, zslab)])
    pltpu.sync_copy(zer_v, acc_in.at[pl.ds(sid * zslab, zslab)])
    plsc.subcore_barrier()

    def body(j, carry):
        pltpu.sync_copy(ones_v, acc_out.at[sidx.at[j]], add=True)
        pltpu.sync_copy(ones_v, acc_in.at[didx.at[j]], add=True)
        return carry

    lax.fori_loop(0, NCHUNK, body, 0)
    plsc.subcore_barrier()
    pltpu.sync_copy(acc_out.at[pl.ds(sid * zslab, zslab)],
                    out_hbm.at[cid, 0, 0, pl.ds(sid * zslab, zslab)])
    pltpu.sync_copy(acc_in.at[pl.ds(sid * zslab, zslab)],
                    out_hbm.at[cid, 1, 0, pl.ds(sid * zslab, zslab)])


def _make_agg(D):
    rpw = NPAD // NS

    @functools.partial(
        pl.kernel,
        out_type=jax.ShapeDtypeStruct((NC, NPAD, D), jnp.float32),
        mesh=_mesh,
        compiler_params=_sc_params,
        scratch_types=[
            pltpu.VMEM((NCHUNK, C), jnp.int32),
            pltpu.VMEM((NCHUNK, C), jnp.int32),
            pltpu.VMEM((C, D), jnp.float32),
            pltpu.VMEM_SHARED((NPAD, D), jnp.float32),
        ],
    )
    def agg(h_hbm, src_hbm, dst_hbm, out_hbm, sidx, didx, buf, acc):
        cid = lax.axis_index("c")
        sid = lax.axis_index("s")
        wid = sid * NC + cid

        pltpu.sync_copy(src_hbm.at[wid], sidx)
        pltpu.sync_copy(dst_hbm.at[wid], didx)

        def zb(i, carry):
            for jj in range(D // 16):
                buf[i, pl.ds(jj * 16, 16)] = jnp.zeros((16,), jnp.float32)
            return carry

        lax.fori_loop(0, C, zb, 0)
        for t in range(rpw // C):
            pltpu.sync_copy(buf, acc.at[pl.ds(sid * rpw + t * C, C)])
        plsc.subcore_barrier()

        def body(j, carry):
            pltpu.sync_copy(h_hbm.at[sidx.at[j]], buf)
            pltpu.sync_copy(buf, acc.at[didx.at[j]], add=True)
            return carry

        lax.fori_loop(0, NCHUNK, body, 0)
        plsc.subcore_barrier()
        pltpu.sync_copy(acc.at[pl.ds(sid * rpw, rpw)],
                        out_hbm.at[cid, pl.ds(sid * rpw, rpw)])

    return agg


_agg128 = _make_agg(128)
_agg64 = _make_agg(64)


def _norms(dp_ref):
    deg_out = dp_ref[0, 0] + dp_ref[1, 0]
    deg_in = dp_ref[0, 1] + dp_ref[1, 1]

    def norm(d):
        return jnp.where(d > 0, lax.rsqrt(jnp.maximum(d, 1.0)), 0.0)[:N]

    return norm(deg_out), norm(deg_in)


def _tc1_body(x_ref, w_ref, dp_ref, o_ref):
    norm_out, _ = _norms(dp_ref)
    h = jnp.dot(x_ref[...], w_ref[...], preferred_element_type=jnp.float32)
    o_ref[...] = h * norm_out


def _tc2_body(aggp_ref, dp_ref, b1_ref, w2_ref, o_ref):
    norm_out, norm_in = _norms(dp_ref)
    agg = aggp_ref[0, :N] + aggp_ref[1, :N]
    x2 = jnp.maximum(agg * norm_in + b1_ref[...][None, :], 0.0)
    h2 = jnp.dot(x2, w2_ref[...], preferred_element_type=jnp.float32)
    o_ref[...] = h2 * norm_out


def _tc3_body(aggp_ref, dp_ref, b2_ref, o_ref):
    _, norm_in = _norms(dp_ref)
    agg = aggp_ref[0, :N] + aggp_ref[1, :N]
    y = jnp.maximum(agg * norm_in + b2_ref[...][None, :], 0.0)
    m = jnp.max(y, axis=1, keepdims=True)
    s = jnp.sum(jnp.exp(y - m), axis=1, keepdims=True)
    o_ref[...] = y - m - jnp.log(s)


_tc1 = pl.pallas_call(
    _tc1_body, out_shape=jax.ShapeDtypeStruct((N, 128), jnp.float32))
_tc2 = pl.pallas_call(
    _tc2_body, out_shape=jax.ShapeDtypeStruct((N, 64), jnp.float32))
_tc3 = pl.pallas_call(
    _tc3_body, out_shape=jax.ShapeDtypeStruct((N, 64), jnp.float32))


def kernel(features, edge_index, W1, b1, W2, b2):
    src = edge_index[0].astype(jnp.int32).reshape(NW, NCHUNK, C)
    dst = edge_index[1].astype(jnp.int32).reshape(NW, NCHUNK, C)
    degp = _deg_kernel(src, dst).reshape(NC, 2, NPAD, 1)
    h1 = _tc1(features, W1, degp)
    agg1 = _agg128(h1, src, dst)
    h2 = _tc2(agg1, degp, b1, W2)
    agg2 = _agg64(h2, src, dst)
    return _tc3(agg2, degp, b2)


# speedup vs baseline: 11.6286x; 1.4509x over previous
import functools

import jax
import jax.numpy as jnp
from jax import lax
from jax.experimental import pallas as pl
from jax.experimental.pallas import tpu as pltpu
from jax.experimental.pallas import tpu_sc as plsc

N = 10000
E = 320000
NC = 2
NS = 16
NW = NC * NS
C = 80
EPW = E // NW
NCHUNK = EPW // C
NPAD = 10240

_mesh = plsc.VectorSubcoreMesh(core_axis_name="c", subcore_axis_name="s")
_sc_params = pltpu.CompilerParams(use_tc_tiling_on_sc=False)


@functools.partial(
    pl.kernel,
    out_type=jax.ShapeDtypeStruct((NC, 2, 1, NPAD), jnp.float32),
    mesh=_mesh,
    compiler_params=_sc_params,
    scratch_types=[
        pltpu.VMEM((NCHUNK, C), jnp.int32),
        pltpu.VMEM((NCHUNK, C), jnp.int32),
        pltpu.VMEM((C,), jnp.float32),
        pltpu.VMEM((NPAD // NS,), jnp.float32),
        pltpu.VMEM_SHARED((NPAD,), jnp.float32),
        pltpu.VMEM_SHARED((NPAD,), jnp.float32),
    ],
)
def _deg_kernel(src_hbm, dst_hbm, out_hbm, sidx, didx, ones_v, zer_v,
                acc_out, acc_in):
    cid = lax.axis_index("c")
    sid = lax.axis_index("s")
    wid = sid * NC + cid
    zslab = NPAD // NS

    for t in range(C // 16):
        ones_v[pl.ds(t * 16, 16)] = jnp.ones((16,), jnp.float32)
    for t in range(zslab // 16):
        zer_v[pl.ds(t * 16, 16)] = jnp.zeros((16,), jnp.float32)

    pltpu.sync_copy(src_hbm.at[wid], sidx)
    pltpu.sync_copy(dst_hbm.at[wid], didx)
    pltpu.sync_copy(zer_v, acc_out.at[pl.ds(sid * zslab, zslab)])
    pltpu.sync_copy(zer_v, acc_in.at[pl.ds(sid * zslab, zslab)])
    plsc.subcore_barrier()

    def body(j, carry):
        pltpu.sync_copy(ones_v, acc_out.at[sidx.at[j]], add=True)
        pltpu.sync_copy(ones_v, acc_in.at[didx.at[j]], add=True)
        return carry

    lax.fori_loop(0, NCHUNK, body, 0)
    plsc.subcore_barrier()
    pltpu.sync_copy(acc_out.at[pl.ds(sid * zslab, zslab)],
                    out_hbm.at[cid, 0, 0, pl.ds(sid * zslab, zslab)])
    pltpu.sync_copy(acc_in.at[pl.ds(sid * zslab, zslab)],
                    out_hbm.at[cid, 1, 0, pl.ds(sid * zslab, zslab)])


def _make_agg(D):
    rpw = NPAD // NS

    @functools.partial(
        pl.kernel,
        out_type=jax.ShapeDtypeStruct((NC, NPAD, D), jnp.float32),
        mesh=_mesh,
        compiler_params=_sc_params,
        scratch_types=[
            pltpu.VMEM((NCHUNK, C), jnp.int32),
            pltpu.VMEM((NCHUNK, C), jnp.int32),
            pltpu.VMEM((2, C, D), jnp.float32),
            pltpu.VMEM_SHARED((NPAD, D), jnp.float32),
            pltpu.SemaphoreType.DMA((2,)),
        ],
    )
    def agg(h_hbm, src_hbm, dst_hbm, out_hbm, sidx, didx, buf, acc, sem):
        cid = lax.axis_index("c")
        sid = lax.axis_index("s")
        wid = sid * NC + cid

        pltpu.sync_copy(src_hbm.at[wid], sidx)
        pltpu.sync_copy(dst_hbm.at[wid], didx)

        def zb(i, carry):
            for jj in range(D // 16):
                buf[0, i, pl.ds(jj * 16, 16)] = jnp.zeros((16,), jnp.float32)
            return carry

        lax.fori_loop(0, C, zb, 0)
        for t in range(rpw // C):
            pltpu.sync_copy(buf.at[0], acc.at[pl.ds(sid * rpw + t * C, C)])
        plsc.subcore_barrier()

        pltpu.async_copy(h_hbm.at[sidx.at[0]], buf.at[0], sem.at[0])

        def body(j, carry):
            p = lax.rem(j, 2)

            @pl.when(j + 1 < NCHUNK)
            def _():
                pltpu.async_copy(h_hbm.at[sidx.at[j + 1]], buf.at[1 - p],
                                 sem.at[1 - p])

            pltpu.make_async_copy(h_hbm.at[sidx.at[j]], buf.at[p],
                                  sem.at[p]).wait()
            pltpu.sync_copy(buf.at[p], acc.at[didx.at[j]], add=True)
            return carry

        lax.fori_loop(0, NCHUNK, body, 0)
        plsc.subcore_barrier()
        pltpu.sync_copy(acc.at[pl.ds(sid * rpw, rpw)],
                        out_hbm.at[cid, pl.ds(sid * rpw, rpw)])

    return agg


_agg128 = _make_agg(128)
_agg64 = _make_agg(64)


def _norms(dp_ref):
    deg_out = dp_ref[0, 0] + dp_ref[1, 0]
    deg_in = dp_ref[0, 1] + dp_ref[1, 1]

    def norm(d):
        return jnp.where(d > 0, lax.rsqrt(jnp.maximum(d, 1.0)), 0.0)[:N]

    return norm(deg_out), norm(deg_in)


def _tc1_body(x_ref, w_ref, dp_ref, o_ref):
    norm_out, _ = _norms(dp_ref)
    h = jnp.dot(x_ref[...], w_ref[...], preferred_element_type=jnp.float32)
    o_ref[...] = h * norm_out


def _tc2_body(aggp_ref, dp_ref, b1_ref, w2_ref, o_ref):
    norm_out, norm_in = _norms(dp_ref)
    agg = aggp_ref[0, :N] + aggp_ref[1, :N]
    x2 = jnp.maximum(agg * norm_in + b1_ref[...][None, :], 0.0)
    h2 = jnp.dot(x2, w2_ref[...], preferred_element_type=jnp.float32)
    o_ref[...] = h2 * norm_out


def _tc3_body(aggp_ref, dp_ref, b2_ref, o_ref):
    _, norm_in = _norms(dp_ref)
    agg = aggp_ref[0, :N] + aggp_ref[1, :N]
    y = jnp.maximum(agg * norm_in + b2_ref[...][None, :], 0.0)
    m = jnp.max(y, axis=1, keepdims=True)
    s = jnp.sum(jnp.exp(y - m), axis=1, keepdims=True)
    o_ref[...] = y - m - jnp.log(s)


_tc1 = pl.pallas_call(
    _tc1_body, out_shape=jax.ShapeDtypeStruct((N, 128), jnp.float32))
_tc2 = pl.pallas_call(
    _tc2_body, out_shape=jax.ShapeDtypeStruct((N, 64), jnp.float32))
_tc3 = pl.pallas_call(
    _tc3_body, out_shape=jax.ShapeDtypeStruct((N, 64), jnp.float32))


def kernel(features, edge_index, W1, b1, W2, b2):
    src = edge_index[0].astype(jnp.int32).reshape(NW, NCHUNK, C)
    dst = edge_index[1].astype(jnp.int32).reshape(NW, NCHUNK, C)
    degp = _deg_kernel(src, dst).reshape(NC, 2, NPAD, 1)
    h1 = _tc1(features, W1, degp)
    agg1 = _agg128(h1, src, dst)
    h2 = _tc2(agg1, degp, b1, W2)
    agg2 = _agg64(h2, src, dst)
    return _tc3(agg2, degp, b2)


# speedup vs baseline: 12.5390x; 1.0783x over previous
import functools

import jax
import jax.numpy as jnp
from jax import lax
from jax.experimental import pallas as pl
from jax.experimental.pallas import tpu as pltpu
from jax.experimental.pallas import tpu_sc as plsc

N = 10000
E = 320000
NC = 2
NS = 16
NW = NC * NS
C = 80
EPW = E // NW
NCHUNK = EPW // C
NPAD = 10240

_mesh = plsc.VectorSubcoreMesh(core_axis_name="c", subcore_axis_name="s")
_sc_params = pltpu.CompilerParams(use_tc_tiling_on_sc=False)


@functools.partial(
    pl.kernel,
    out_type=jax.ShapeDtypeStruct((NC, 2, 1, NPAD), jnp.float32),
    mesh=_mesh,
    compiler_params=_sc_params,
    scratch_types=[
        pltpu.VMEM((NCHUNK, C), jnp.int32),
        pltpu.VMEM((NCHUNK, C), jnp.int32),
        pltpu.VMEM((C,), jnp.float32),
        pltpu.VMEM((NPAD // NS,), jnp.float32),
        pltpu.VMEM_SHARED((NPAD,), jnp.float32),
        pltpu.VMEM_SHARED((NPAD,), jnp.float32),
        pltpu.SemaphoreType.DMA,
    ],
)
def _deg_kernel(src_hbm, dst_hbm, out_hbm, sidx, didx, ones_v, zer_v,
                acc_out, acc_in, sem):
    cid = lax.axis_index("c")
    sid = lax.axis_index("s")
    wid = sid * NC + cid
    zslab = NPAD // NS

    for t in range(C // 16):
        ones_v[pl.ds(t * 16, 16)] = jnp.ones((16,), jnp.float32)
    for t in range(zslab // 16):
        zer_v[pl.ds(t * 16, 16)] = jnp.zeros((16,), jnp.float32)

    pltpu.sync_copy(src_hbm.at[wid], sidx)
    pltpu.sync_copy(dst_hbm.at[wid], didx)
    pltpu.sync_copy(zer_v, acc_out.at[pl.ds(sid * zslab, zslab)])
    pltpu.sync_copy(zer_v, acc_in.at[pl.ds(sid * zslab, zslab)])
    plsc.subcore_barrier()

    lag = 8

    def body(j, carry):
        pltpu.async_copy(ones_v, acc_out.at[sidx.at[j]], sem, add=True)
        pltpu.async_copy(ones_v, acc_in.at[didx.at[j]], sem, add=True)

        @pl.when(j >= lag)
        def _():
            pltpu.make_async_copy(ones_v, acc_out.at[sidx.at[0]], sem).wait()
            pltpu.make_async_copy(ones_v, acc_out.at[sidx.at[0]], sem).wait()

        return carry

    lax.fori_loop(0, NCHUNK, body, 0)
    for _ in range(2 * lag):
        pltpu.make_async_copy(ones_v, acc_out.at[sidx.at[0]], sem).wait()
    plsc.subcore_barrier()
    pltpu.sync_copy(acc_out.at[pl.ds(sid * zslab, zslab)],
                    out_hbm.at[cid, 0, 0, pl.ds(sid * zslab, zslab)])
    pltpu.sync_copy(acc_in.at[pl.ds(sid * zslab, zslab)],
                    out_hbm.at[cid, 1, 0, pl.ds(sid * zslab, zslab)])


def _make_agg(D, nbuf):
    rpw = NPAD // NS

    @functools.partial(
        pl.kernel,
        out_type=jax.ShapeDtypeStruct((NC, NPAD, D), jnp.float32),
        mesh=_mesh,
        compiler_params=_sc_params,
        scratch_types=[
            pltpu.VMEM((NCHUNK, C), jnp.int32),
            pltpu.VMEM((NCHUNK, C), jnp.int32),
            pltpu.VMEM((nbuf, C, D), jnp.float32),
            pltpu.VMEM_SHARED((NPAD, D), jnp.float32),
            pltpu.SemaphoreType.DMA((nbuf,)),
            pltpu.SemaphoreType.DMA((nbuf,)),
        ],
    )
    def agg(h_hbm, src_hbm, dst_hbm, out_hbm, sidx, didx, buf, acc,
            semg, sems):
        cid = lax.axis_index("c")
        sid = lax.axis_index("s")
        wid = sid * NC + cid

        pltpu.sync_copy(src_hbm.at[wid], sidx)
        pltpu.sync_copy(dst_hbm.at[wid], didx)

        def zb(i, carry):
            for jj in range(D // 16):
                buf[0, i, pl.ds(jj * 16, 16)] = jnp.zeros((16,), jnp.float32)
            return carry

        lax.fori_loop(0, C, zb, 0)
        for t in range(rpw // C):
            pltpu.sync_copy(buf.at[0], acc.at[pl.ds(sid * rpw + t * C, C)])
        plsc.subcore_barrier()

        pltpu.async_copy(h_hbm.at[sidx.at[0]], buf.at[0], semg.at[0])
        pltpu.async_copy(h_hbm.at[sidx.at[1]], buf.at[1], semg.at[1])

        def body(j, carry):
            p = lax.rem(j, nbuf)
            pltpu.make_async_copy(h_hbm.at[sidx.at[j]], buf.at[p],
                                  semg.at[p]).wait()
            pltpu.async_copy(buf.at[p], acc.at[didx.at[j]], sems.at[p],
                             add=True)

            @pl.when(j + 2 < NCHUNK)
            def _():
                q = lax.rem(j + 2, nbuf)

                @pl.when(j >= nbuf - 2)
                def _():
                    pltpu.make_async_copy(
                        buf.at[q], acc.at[didx.at[0]], sems.at[q]).wait()

                pltpu.async_copy(h_hbm.at[sidx.at[j + 2]], buf.at[q],
                                 semg.at[q])

            return carry

        lax.fori_loop(0, NCHUNK, body, 0)
        for k in range(NCHUNK - nbuf, NCHUNK):
            pltpu.make_async_copy(buf.at[k % nbuf], acc.at[didx.at[0]],
                                  sems.at[k % nbuf]).wait()
        plsc.subcore_barrier()
        pltpu.sync_copy(acc.at[pl.ds(sid * rpw, rpw)],
                        out_hbm.at[cid, pl.ds(sid * rpw, rpw)])

    return agg


_agg128 = _make_agg(128, 2)
_agg64 = _make_agg(64, 4)


RB = 1000


def _norms(dp_ref):
    deg_out = dp_ref[0, 0] + dp_ref[1, 0]
    deg_in = dp_ref[0, 1] + dp_ref[1, 1]

    def norm(d):
        return jnp.where(d > 0, lax.rsqrt(jnp.maximum(d, 1.0)), 0.0)

    return norm(deg_out), norm(deg_in)


def _tc1_body(x_ref, w_ref, dp_ref, o_ref):
    norm_out, _ = _norms(dp_ref)
    h = jnp.dot(x_ref[...], w_ref[...], preferred_element_type=jnp.float32)
    o_ref[...] = h * norm_out


def _tc2_body(aggp_ref, dp_ref, b1_ref, w2_ref, o_ref):
    norm_out, norm_in = _norms(dp_ref)
    agg = aggp_ref[0] + aggp_ref[1]
    x2 = jnp.maximum(agg * norm_in + b1_ref[...], 0.0)
    h2 = jnp.dot(x2, w2_ref[...], preferred_element_type=jnp.float32)
    o_ref[...] = h2 * norm_out


def _tc3_body(aggp_ref, dp_ref, b2_ref, o_ref):
    _, norm_in = _norms(dp_ref)
    agg = aggp_ref[0] + aggp_ref[1]
    y = jnp.maximum(agg * norm_in + b2_ref[...], 0.0)
    m = jnp.max(y, axis=1, keepdims=True)
    s = jnp.sum(jnp.exp(y - m), axis=1, keepdims=True)
    o_ref[...] = y - m - jnp.log(s)


_dp_spec = pl.BlockSpec((NC, 2, RB, 1), lambda i: (0, 0, i, 0))

_tc1 = pl.pallas_call(
    _tc1_body,
    grid=(N // RB,),
    in_specs=[pl.BlockSpec((RB, 128), lambda i: (i, 0)),
              pl.BlockSpec((128, 128), lambda i: (0, 0)),
              _dp_spec],
    out_specs=pl.BlockSpec((RB, 128), lambda i: (i, 0)),
    out_shape=jax.ShapeDtypeStruct((N, 128), jnp.float32))
_tc2 = pl.pallas_call(
    _tc2_body,
    grid=(N // RB,),
    in_specs=[pl.BlockSpec((NC, RB, 128), lambda i: (0, i, 0)),
              _dp_spec,
              pl.BlockSpec((1, 128), lambda i: (0, 0)),
              pl.BlockSpec((128, 64), lambda i: (0, 0))],
    out_specs=pl.BlockSpec((RB, 64), lambda i: (i, 0)),
    out_shape=jax.ShapeDtypeStruct((N, 64), jnp.float32))
_tc3 = pl.pallas_call(
    _tc3_body,
    grid=(N // RB,),
    in_specs=[pl.BlockSpec((NC, RB, 64), lambda i: (0, i, 0)),
              _dp_spec,
              pl.BlockSpec((1, 64), lambda i: (0, 0))],
    out_specs=pl.BlockSpec((RB, 64), lambda i: (i, 0)),
    out_shape=jax.ShapeDtypeStruct((N, 64), jnp.float32))


def kernel(features, edge_index, W1, b1, W2, b2):
    src = edge_index[0].astype(jnp.int32).reshape(NW, NCHUNK, C)
    dst = edge_index[1].astype(jnp.int32).reshape(NW, NCHUNK, C)
    degp = _deg_kernel(src, dst).reshape(NC, 2, NPAD, 1)
    h1 = _tc1(features, W1, degp)
    agg1 = _agg128(h1, src, dst)
    h2 = _tc2(agg1, degp, b1.reshape(1, 128), W2)
    agg2 = _agg64(h2, src, dst)
    return _tc3(agg2, degp, b2.reshape(1, 64))


# speedup vs baseline: 13.1130x; 1.0458x over previous
import functools

import jax
import jax.numpy as jnp
from jax import lax
from jax.experimental import pallas as pl
from jax.experimental.pallas import tpu as pltpu
from jax.experimental.pallas import tpu_sc as plsc

N = 10000
E = 320000
NC = 2
NS = 16
NW = NC * NS
C = 80
EPW = E // NW
NCHUNK = EPW // C
NPAD = 10240

_mesh = plsc.VectorSubcoreMesh(core_axis_name="c", subcore_axis_name="s")
_sc_params = pltpu.CompilerParams(use_tc_tiling_on_sc=False)


@functools.partial(
    pl.kernel,
    out_type=jax.ShapeDtypeStruct((NC, 2, 1, NPAD), jnp.float32),
    mesh=_mesh,
    compiler_params=_sc_params,
    scratch_types=[
        pltpu.VMEM((NCHUNK, C), jnp.int32),
        pltpu.VMEM((NCHUNK, C), jnp.int32),
        pltpu.VMEM((C,), jnp.float32),
        pltpu.VMEM((NPAD // NS,), jnp.float32),
        pltpu.VMEM_SHARED((NPAD,), jnp.float32),
        pltpu.VMEM_SHARED((NPAD,), jnp.float32),
        pltpu.SemaphoreType.DMA,
    ],
)
def _deg_kernel(src_hbm, dst_hbm, out_hbm, sidx, didx, ones_v, zer_v,
                acc_out, acc_in, sem):
    cid = lax.axis_index("c")
    sid = lax.axis_index("s")
    wid = sid * NC + cid
    zslab = NPAD // NS

    for t in range(C // 16):
        ones_v[pl.ds(t * 16, 16)] = jnp.ones((16,), jnp.float32)
    for t in range(zslab // 16):
        zer_v[pl.ds(t * 16, 16)] = jnp.zeros((16,), jnp.float32)

    pltpu.sync_copy(src_hbm.at[wid], sidx)
    pltpu.sync_copy(dst_hbm.at[wid], didx)
    pltpu.sync_copy(zer_v, acc_out.at[pl.ds(sid * zslab, zslab)])
    pltpu.sync_copy(zer_v, acc_in.at[pl.ds(sid * zslab, zslab)])
    plsc.subcore_barrier()

    lag = 8

    def body(j, carry):
        pltpu.async_copy(ones_v, acc_out.at[sidx.at[j]], sem, add=True)
        pltpu.async_copy(ones_v, acc_in.at[didx.at[j]], sem, add=True)

        @pl.when(j >= lag)
        def _():
            pltpu.make_async_copy(ones_v, acc_out.at[sidx.at[0]], sem).wait()
            pltpu.make_async_copy(ones_v, acc_out.at[sidx.at[0]], sem).wait()

        return carry

    lax.fori_loop(0, NCHUNK, body, 0)
    for _ in range(2 * lag):
        pltpu.make_async_copy(ones_v, acc_out.at[sidx.at[0]], sem).wait()
    plsc.subcore_barrier()
    pltpu.sync_copy(acc_out.at[pl.ds(sid * zslab, zslab)],
                    out_hbm.at[cid, 0, 0, pl.ds(sid * zslab, zslab)])
    pltpu.sync_copy(acc_in.at[pl.ds(sid * zslab, zslab)],
                    out_hbm.at[cid, 1, 0, pl.ds(sid * zslab, zslab)])


def _make_agg(D, nbuf, dtype=jnp.float32):
    rpw = NPAD // NS

    @functools.partial(
        pl.kernel,
        out_type=jax.ShapeDtypeStruct((NC, NPAD, D), dtype),
        mesh=_mesh,
        compiler_params=_sc_params,
        scratch_types=[
            pltpu.VMEM((NCHUNK, C), jnp.int32),
            pltpu.VMEM((NCHUNK, C), jnp.int32),
            pltpu.VMEM((nbuf, C, D), dtype),
            pltpu.VMEM_SHARED((NPAD, D), dtype),
            pltpu.SemaphoreType.DMA((nbuf,)),
            pltpu.SemaphoreType.DMA((nbuf,)),
        ],
    )
    def agg(h_hbm, src_hbm, dst_hbm, out_hbm, sidx, didx, buf, acc,
            semg, sems):
        cid = lax.axis_index("c")
        sid = lax.axis_index("s")
        wid = sid * NC + cid

        pltpu.sync_copy(src_hbm.at[wid], sidx)
        pltpu.sync_copy(dst_hbm.at[wid], didx)

        lanes = 32 if dtype == jnp.bfloat16 else 16

        def zb(i, carry):
            for jj in range(D // lanes):
                buf[0, i, pl.ds(jj * lanes, lanes)] = jnp.zeros((lanes,), dtype)
            return carry

        lax.fori_loop(0, C, zb, 0)
        for t in range(rpw // C):
            pltpu.sync_copy(buf.at[0], acc.at[pl.ds(sid * rpw + t * C, C)])
        plsc.subcore_barrier()

        pltpu.async_copy(h_hbm.at[sidx.at[0]], buf.at[0], semg.at[0])
        pltpu.async_copy(h_hbm.at[sidx.at[1]], buf.at[1], semg.at[1])

        def body(j, carry):
            p = lax.rem(j, nbuf)
            pltpu.make_async_copy(h_hbm.at[sidx.at[j]], buf.at[p],
                                  semg.at[p]).wait()
            pltpu.async_copy(buf.at[p], acc.at[didx.at[j]], sems.at[p],
                             add=True)

            @pl.when(j + 2 < NCHUNK)
            def _():
                q = lax.rem(j + 2, nbuf)

                @pl.when(j >= nbuf - 2)
                def _():
                    pltpu.make_async_copy(
                        buf.at[q], acc.at[didx.at[0]], sems.at[q]).wait()

                pltpu.async_copy(h_hbm.at[sidx.at[j + 2]], buf.at[q],
                                 semg.at[q])

            return carry

        lax.fori_loop(0, NCHUNK, body, 0)
        for k in range(NCHUNK - nbuf, NCHUNK):
            pltpu.make_async_copy(buf.at[k % nbuf], acc.at[didx.at[0]],
                                  sems.at[k % nbuf]).wait()
        plsc.subcore_barrier()
        pltpu.sync_copy(acc.at[pl.ds(sid * rpw, rpw)],
                        out_hbm.at[cid, pl.ds(sid * rpw, rpw)])

    return agg


_agg128 = _make_agg(128, 4, jnp.bfloat16)
_agg64 = _make_agg(64, 4)


RB = 1000


def _norms(dp_ref):
    deg_out = dp_ref[0, 0] + dp_ref[1, 0]
    deg_in = dp_ref[0, 1] + dp_ref[1, 1]

    def norm(d):
        return jnp.where(d > 0, lax.rsqrt(jnp.maximum(d, 1.0)), 0.0)

    return norm(deg_out), norm(deg_in)


def _tc1_body(x_ref, w_ref, dp_ref, o_ref):
    norm_out, _ = _norms(dp_ref)
    h = jnp.dot(x_ref[...], w_ref[...], preferred_element_type=jnp.float32)
    o_ref[...] = (h * norm_out).astype(jnp.bfloat16)


def _tc2_body(aggp_ref, dp_ref, b1_ref, w2_ref, o_ref):
    norm_out, norm_in = _norms(dp_ref)
    agg = (aggp_ref[0].astype(jnp.float32) +
           aggp_ref[1].astype(jnp.float32))
    x2 = jnp.maximum(agg * norm_in + b1_ref[...], 0.0)
    h2 = jnp.dot(x2, w2_ref[...], preferred_element_type=jnp.float32)
    o_ref[...] = h2 * norm_out


def _tc3_body(aggp_ref, dp_ref, b2_ref, o_ref):
    _, norm_in = _norms(dp_ref)
    agg = aggp_ref[0] + aggp_ref[1]
    y = jnp.maximum(agg * norm_in + b2_ref[...], 0.0)
    m = jnp.max(y, axis=1, keepdims=True)
    s = jnp.sum(jnp.exp(y - m), axis=1, keepdims=True)
    o_ref[...] = y - m - jnp.log(s)


_dp_spec = pl.BlockSpec((NC, 2, RB, 1), lambda i: (0, 0, i, 0))

_tc1 = pl.pallas_call(
    _tc1_body,
    grid=(N // RB,),
    in_specs=[pl.BlockSpec((RB, 128), lambda i: (i, 0)),
              pl.BlockSpec((128, 128), lambda i: (0, 0)),
              _dp_spec],
    out_specs=pl.BlockSpec((RB, 128), lambda i: (i, 0)),
    out_shape=jax.ShapeDtypeStruct((N, 128), jnp.bfloat16))
_tc2 = pl.pallas_call(
    _tc2_body,
    grid=(N // RB,),
    in_specs=[pl.BlockSpec((NC, RB, 128), lambda i: (0, i, 0)),
              _dp_spec,
              pl.BlockSpec((1, 128), lambda i: (0, 0)),
              pl.BlockSpec((128, 64), lambda i: (0, 0))],
    out_specs=pl.BlockSpec((RB, 64), lambda i: (i, 0)),
    out_shape=jax.ShapeDtypeStruct((N, 64), jnp.float32))
_tc3 = pl.pallas_call(
    _tc3_body,
    grid=(N // RB,),
    in_specs=[pl.BlockSpec((NC, RB, 64), lambda i: (0, i, 0)),
              _dp_spec,
              pl.BlockSpec((1, 64), lambda i: (0, 0))],
    out_specs=pl.BlockSpec((RB, 64), lambda i: (i, 0)),
    out_shape=jax.ShapeDtypeStruct((N, 64), jnp.float32))


def kernel(features, edge_index, W1, b1, W2, b2):
    src = edge_index[0].astype(jnp.int32).reshape(NW, NCHUNK, C)
    dst = edge_index[1].astype(jnp.int32).reshape(NW, NCHUNK, C)
    degp = _deg_kernel(src, dst).reshape(NC, 2, NPAD, 1)
    h1 = _tc1(features, W1, degp)
    agg1 = _agg128(h1, src, dst)
    h2 = _tc2(agg1, degp, b1.reshape(1, 128), W2)
    agg2 = _agg64(h2, src, dst)
    return _tc3(agg2, degp, b2.reshape(1, 64))
